# Initial kernel scaffold; baseline (speedup 1.0000x reference)
import jax
import jax.numpy as jnp
from jax.experimental import pallas as pl


def kernel(x, edge_index, a):
    raise NotImplementedError("write your pallas kernel here")


# speedup vs baseline: 6.0317x; 6.0317x over previous
import functools

import jax
import jax.numpy as jnp
from jax import lax
from jax.experimental import pallas as pl
from jax.experimental.pallas import tpu as pltpu
from jax.experimental.pallas import tpu_sc as plsc

_N_NODES = 10000
_D = 128
_PAD = 10240
_CHUNK = 128
_NTILES = 32
_PER = 80
_PAD_EDGES = _NTILES * _PER * _CHUNK
_SLICE = _PAD // 16

_mesh = plsc.VectorSubcoreMesh(core_axis_name="c", subcore_axis_name="s")


@functools.partial(
    pl.kernel,
    out_type=(
        jax.ShapeDtypeStruct((_PAD,), jnp.float32),
        jax.ShapeDtypeStruct((_PAD,), jnp.float32),
    ),
    mesh=_mesh,
    scratch_types=[
        pltpu.VMEM((_PER, _CHUNK), jnp.int32),
        pltpu.VMEM((1, _CHUNK), jnp.float32),
        pltpu.VMEM_SHARED((_PAD,), jnp.float32),
        pltpu.SemaphoreType.DMA,
        pltpu.SemaphoreType.DMA,
    ],
)
def _degree_hist(edges_hbm, ones_hbm, zeros_hbm, out0, out1,
                 idx_v, ones_v, hist_sh, sem_ld, sem_sc):
    cid = lax.axis_index("c")
    sid = lax.axis_index("s")
    w = cid * 16 + sid

    pltpu.sync_copy(ones_hbm, ones_v)
    pltpu.sync_copy(zeros_hbm, hist_sh.at[pl.ds(sid * _SLICE, _SLICE)])
    ld = pltpu.async_copy(edges_hbm.at[w], idx_v, sem_ld)

    plsc.subcore_barrier()
    ld.wait()

    @pl.loop(0, _PER)
    def _(j):
        pltpu.async_copy(ones_v.at[0], hist_sh.at[idx_v.at[j]], sem_sc,
                         add=True)

    @pl.loop(0, _PER)
    def _(j):
        pltpu.make_async_copy(ones_v.at[0], hist_sh.at[idx_v.at[j]],
                              sem_sc).wait()

    plsc.subcore_barrier()

    sl = pl.ds(sid * _SLICE, _SLICE)

    @pl.when(cid == 0)
    def _():
        pltpu.sync_copy(hist_sh.at[sl], out0.at[sl])

    @pl.when(cid == 1)
    def _():
        pltpu.sync_copy(hist_sh.at[sl], out1.at[sl])


_ROWS = 1024


def _norm_block(x_ref, h0_ref, h1_ref, a_ref, o_ref):
    deg = h0_ref[...] + h1_ref[...]
    a = a_ref[0]
    n = jnp.exp(a * jnp.log(deg + 1.0))
    o_ref[...] = x_ref[...] / n


def _normalize(x, h0, h1, a_arr):
    return pl.pallas_call(
        _norm_block,
        grid=(pl.cdiv(_N_NODES, _ROWS),),
        in_specs=[
            pl.BlockSpec((_ROWS, _D), lambda i: (i, 0)),
            pl.BlockSpec((_ROWS, 1), lambda i: (i, 0)),
            pl.BlockSpec((_ROWS, 1), lambda i: (i, 0)),
            pl.BlockSpec(memory_space=pltpu.SMEM),
        ],
        out_specs=pl.BlockSpec((_ROWS, _D), lambda i: (i, 0)),
        out_shape=jax.ShapeDtypeStruct((_N_NODES, _D), jnp.float32),
    )(x, h0, h1, a_arr)


def kernel(x, edge_index, a):
    src = edge_index[0]
    pad = jnp.full((_PAD_EDGES - src.shape[0],), _N_NODES, jnp.int32)
    src = jnp.concatenate([src, pad]).reshape(_NTILES, _PER, _CHUNK)
    ones = jnp.ones((1, _CHUNK), jnp.float32)
    zeros = jnp.zeros((_SLICE,), jnp.float32)
    h0, h1 = _degree_hist(src, ones, zeros)
    a_arr = jnp.asarray(a, jnp.float32).reshape(1)
    return _normalize(x, h0.reshape(_PAD, 1), h1.reshape(_PAD, 1), a_arr)


# speedup vs baseline: 6.4211x; 1.0646x over previous
import functools

import jax
import jax.numpy as jnp
from jax import lax
from jax.experimental import pallas as pl
from jax.experimental.pallas import tpu as pltpu
from jax.experimental.pallas import tpu_sc as plsc

_N_NODES = 10000
_D = 128
_PAD = 10240
_CHUNK = 125
_NTILES = 32
_PER = 80
_SLICE = _PAD // 16

_mesh = plsc.VectorSubcoreMesh(core_axis_name="c", subcore_axis_name="s")


@functools.partial(
    pl.kernel,
    out_type=(
        jax.ShapeDtypeStruct((_PAD,), jnp.float32),
        jax.ShapeDtypeStruct((_PAD,), jnp.float32),
    ),
    mesh=_mesh,
    scratch_types=[
        pltpu.VMEM((_PER, _CHUNK), jnp.int32),
        pltpu.VMEM((1, _CHUNK), jnp.float32),
        pltpu.VMEM_SHARED((_PAD,), jnp.float32),
        pltpu.SemaphoreType.DMA,
        pltpu.SemaphoreType.DMA,
    ],
)
def _degree_hist(edges_hbm, ones_hbm, zeros_hbm, out0, out1,
                 idx_v, ones_v, hist_sh, sem_ld, sem_sc):
    cid = lax.axis_index("c")
    sid = lax.axis_index("s")
    w = cid * 16 + sid

    pltpu.sync_copy(ones_hbm, ones_v)
    pltpu.sync_copy(zeros_hbm, hist_sh.at[pl.ds(sid * _SLICE, _SLICE)])
    ld = pltpu.async_copy(edges_hbm.at[w], idx_v, sem_ld)

    plsc.subcore_barrier()
    ld.wait()

    @pl.loop(0, _PER)
    def _(j):
        pltpu.async_copy(ones_v.at[0], hist_sh.at[idx_v.at[j]], sem_sc,
                         add=True)

    @pl.loop(0, _PER)
    def _(j):
        pltpu.make_async_copy(ones_v.at[0], hist_sh.at[idx_v.at[j]],
                              sem_sc).wait()

    plsc.subcore_barrier()

    sl = pl.ds(sid * _SLICE, _SLICE)

    @pl.when(cid == 0)
    def _():
        pltpu.sync_copy(hist_sh.at[sl], out0.at[sl])

    @pl.when(cid == 1)
    def _():
        pltpu.sync_copy(hist_sh.at[sl], out1.at[sl])


_ROWS = 1024


def _norm_block(x_ref, h0_ref, h1_ref, a_ref, o_ref):
    deg = h0_ref[...] + h1_ref[...]
    a = a_ref[0]
    n = jnp.exp(a * jnp.log(deg + 1.0))
    o_ref[...] = x_ref[...] / n


def _normalize(x, h0, h1, a_arr):
    return pl.pallas_call(
        _norm_block,
        grid=(pl.cdiv(_N_NODES, _ROWS),),
        in_specs=[
            pl.BlockSpec((_ROWS, _D), lambda i: (i, 0)),
            pl.BlockSpec((_ROWS, 1), lambda i: (i, 0)),
            pl.BlockSpec((_ROWS, 1), lambda i: (i, 0)),
            pl.BlockSpec(memory_space=pltpu.SMEM),
        ],
        out_specs=pl.BlockSpec((_ROWS, _D), lambda i: (i, 0)),
        out_shape=jax.ShapeDtypeStruct((_N_NODES, _D), jnp.float32),
    )(x, h0, h1, a_arr)


def kernel(x, edge_index, a):
    src = edge_index[0].reshape(_NTILES, _PER, _CHUNK)
    ones = jnp.ones((1, _CHUNK), jnp.float32)
    zeros = jnp.zeros((_SLICE,), jnp.float32)
    h0, h1 = _degree_hist(src, ones, zeros)
    a_arr = jnp.asarray(a, jnp.float32).reshape(1)
    return _normalize(x, h0.reshape(_PAD, 1), h1.reshape(_PAD, 1), a_arr)


# speedup vs baseline: 7.5714x; 1.1791x over previous
import functools

import jax
import jax.numpy as jnp
from jax import lax
from jax.experimental import pallas as pl
from jax.experimental.pallas import tpu as pltpu
from jax.experimental.pallas import tpu_sc as plsc

_N_NODES = 10000
_D = 128
_PAD = 10240
_CHUNK = 125
_NTILES = 32
_PER = 80
_SLICE = _PAD // 16

_mesh = plsc.VectorSubcoreMesh(core_axis_name="c", subcore_axis_name="s")


@functools.partial(
    pl.kernel,
    out_type=(
        jax.ShapeDtypeStruct((_PAD,), jnp.float32),
        jax.ShapeDtypeStruct((_PAD,), jnp.float32),
    ),
    mesh=_mesh,
    scratch_types=[
        pltpu.VMEM((_PER, _CHUNK), jnp.int32),
        pltpu.VMEM((1, _CHUNK), jnp.float32),
        pltpu.VMEM_SHARED((_PAD,), jnp.float32),
        pltpu.SemaphoreType.DMA,
        pltpu.SemaphoreType.DMA,
    ],
)
def _degree_hist(edges_hbm, ones_hbm, zeros_hbm, out0, out1,
                 idx_v, ones_v, hist_sh, sem_ld, sem_sc):
    cid = lax.axis_index("c")
    sid = lax.axis_index("s")
    w = cid * 16 + sid

    pltpu.sync_copy(ones_hbm, ones_v)
    pltpu.sync_copy(zeros_hbm, hist_sh.at[pl.ds(sid * _SLICE, _SLICE)])
    ld = pltpu.async_copy(edges_hbm.at[w], idx_v, sem_ld)

    plsc.subcore_barrier()
    ld.wait()

    @pl.loop(0, _PER)
    def _(j):
        pltpu.async_copy(ones_v.at[0], hist_sh.at[idx_v.at[j]], sem_sc,
                         add=True)

    @pl.loop(0, _PER)
    def _(j):
        pltpu.make_async_copy(ones_v.at[0], hist_sh.at[idx_v.at[j]],
                              sem_sc).wait()

    plsc.subcore_barrier()

    sl = pl.ds(sid * _SLICE, _SLICE)

    @pl.when(cid == 0)
    def _():
        pltpu.sync_copy(hist_sh.at[sl], out0.at[sl])

    @pl.when(cid == 1)
    def _():
        pltpu.sync_copy(hist_sh.at[sl], out1.at[sl])


_ROWS = 1024


def _norm_block(x_ref, h0_ref, h1_ref, a_ref, o_ref):
    deg = h0_ref[...] + h1_ref[...]
    a = a_ref[0]
    n = jnp.exp(a * jnp.log(deg + 1.0))
    n_col = jnp.reshape(n, (_ROWS, 1))
    o_ref[...] = x_ref[...] / n_col


def _normalize(x, h0, h1, a_arr):
    return pl.pallas_call(
        _norm_block,
        grid=(pl.cdiv(_N_NODES, _ROWS),),
        in_specs=[
            pl.BlockSpec((_ROWS, _D), lambda i: (i, 0)),
            pl.BlockSpec((1, _ROWS), lambda i: (0, i)),
            pl.BlockSpec((1, _ROWS), lambda i: (0, i)),
            pl.BlockSpec(memory_space=pltpu.SMEM),
        ],
        out_specs=pl.BlockSpec((_ROWS, _D), lambda i: (i, 0)),
        out_shape=jax.ShapeDtypeStruct((_N_NODES, _D), jnp.float32),
    )(x, h0, h1, a_arr)


def kernel(x, edge_index, a):
    src = edge_index[0].reshape(_NTILES, _PER, _CHUNK)
    ones = jnp.ones((1, _CHUNK), jnp.float32)
    zeros = jnp.zeros((_SLICE,), jnp.float32)
    h0, h1 = _degree_hist(src, ones, zeros)
    a_arr = jnp.asarray(a, jnp.float32).reshape(1)
    return _normalize(x, h0.reshape(1, _PAD), h1.reshape(1, _PAD), a_arr)


# speedup vs baseline: 7.5874x; 1.0021x over previous
import functools

import jax
import jax.numpy as jnp
from jax import lax
from jax.experimental import pallas as pl
from jax.experimental.pallas import tpu as pltpu
from jax.experimental.pallas import tpu_sc as plsc

_N_NODES = 10000
_D = 128
_PAD = 10240
_CHUNK = 125
_NTILES = 32
_PER = 80
_SLICE = _PAD // 16

_mesh = plsc.VectorSubcoreMesh(core_axis_name="c", subcore_axis_name="s")


@functools.partial(
    pl.kernel,
    out_type=(
        jax.ShapeDtypeStruct((_PAD,), jnp.float32),
        jax.ShapeDtypeStruct((_PAD,), jnp.float32),
    ),
    mesh=_mesh,
    scratch_types=[
        pltpu.VMEM((_PER, _CHUNK), jnp.int32),
        pltpu.VMEM((1, _CHUNK), jnp.float32),
        pltpu.VMEM_SHARED((_PAD,), jnp.float32),
        pltpu.SemaphoreType.DMA,
        pltpu.SemaphoreType.DMA,
    ],
)
def _degree_hist(edges_hbm, ones_hbm, zeros_hbm, out0, out1,
                 idx_v, ones_v, hist_sh, sem_ld, sem_sc):
    cid = lax.axis_index("c")
    sid = lax.axis_index("s")
    w = cid * 16 + sid

    pltpu.sync_copy(ones_hbm, ones_v)
    pltpu.sync_copy(zeros_hbm, hist_sh.at[pl.ds(sid * _SLICE, _SLICE)])
    ld = pltpu.async_copy(edges_hbm.at[w], idx_v, sem_ld)

    plsc.subcore_barrier()
    ld.wait()

    @pl.loop(0, _PER)
    def _(j):
        pltpu.async_copy(ones_v.at[0], hist_sh.at[idx_v.at[j]], sem_sc,
                         add=True)

    @pl.loop(0, _PER)
    def _(j):
        pltpu.make_async_copy(ones_v.at[0], hist_sh.at[idx_v.at[j]],
                              sem_sc).wait()

    plsc.subcore_barrier()

    sl = pl.ds(sid * _SLICE, _SLICE)

    @pl.when(cid == 0)
    def _():
        pltpu.sync_copy(hist_sh.at[sl], out0.at[sl])

    @pl.when(cid == 1)
    def _():
        pltpu.sync_copy(hist_sh.at[sl], out1.at[sl])


_ROWS = 1024


def _norm_block(x_ref, h0_ref, h1_ref, a_ref, o_ref):
    deg = h0_ref[...] + h1_ref[...]
    a = a_ref[0]
    inv_n = jnp.exp(-a * jnp.log(deg + 1.0))
    n_col = jnp.reshape(inv_n, (_ROWS, 1))
    o_ref[...] = x_ref[...] * n_col


def _normalize(x, h0, h1, a_arr):
    return pl.pallas_call(
        _norm_block,
        grid=(pl.cdiv(_N_NODES, _ROWS),),
        in_specs=[
            pl.BlockSpec((_ROWS, _D), lambda i: (i, 0)),
            pl.BlockSpec((1, _ROWS), lambda i: (0, i)),
            pl.BlockSpec((1, _ROWS), lambda i: (0, i)),
            pl.BlockSpec(memory_space=pltpu.SMEM),
        ],
        out_specs=pl.BlockSpec((_ROWS, _D), lambda i: (i, 0)),
        out_shape=jax.ShapeDtypeStruct((_N_NODES, _D), jnp.float32),
    )(x, h0, h1, a_arr)


def kernel(x, edge_index, a):
    src = edge_index[0].reshape(_NTILES, _PER, _CHUNK)
    ones = jnp.ones((1, _CHUNK), jnp.float32)
    zeros = jnp.zeros((_SLICE,), jnp.float32)
    h0, h1 = _degree_hist(src, ones, zeros)
    a_arr = jnp.asarray(a, jnp.float32).reshape(1)
    return _normalize(x, h0.reshape(1, _PAD), h1.reshape(1, _PAD), a_arr)


# speedup vs baseline: 7.5973x; 1.0013x over previous
import functools

import jax
import jax.numpy as jnp
from jax import lax
from jax.experimental import pallas as pl
from jax.experimental.pallas import tpu as pltpu
from jax.experimental.pallas import tpu_sc as plsc

_N_NODES = 10000
_D = 128
_PAD = 10240
_CHUNK = 125
_NTILES = 32
_PER = 80
_SLICE = _PAD // 16

_mesh = plsc.VectorSubcoreMesh(core_axis_name="c", subcore_axis_name="s")


@functools.partial(
    pl.kernel,
    out_type=(
        jax.ShapeDtypeStruct((_PAD,), jnp.float32),
        jax.ShapeDtypeStruct((_PAD,), jnp.float32),
    ),
    mesh=_mesh,
    scratch_types=[
        pltpu.VMEM((_PER, _CHUNK), jnp.int32),
        pltpu.VMEM((1, _CHUNK), jnp.float32),
        pltpu.VMEM_SHARED((_PAD,), jnp.float32),
        pltpu.SemaphoreType.DMA,
        pltpu.SemaphoreType.DMA,
    ],
)
def _degree_hist(edges_hbm, ones_hbm, zeros_hbm, out0, out1,
                 idx_v, ones_v, hist_sh, sem_ld, sem_sc):
    cid = lax.axis_index("c")
    sid = lax.axis_index("s")
    w = cid * 16 + sid

    pltpu.sync_copy(ones_hbm, ones_v)
    pltpu.sync_copy(zeros_hbm, hist_sh.at[pl.ds(sid * _SLICE, _SLICE)])
    ld = pltpu.async_copy(edges_hbm.at[w], idx_v, sem_ld)

    plsc.subcore_barrier()
    ld.wait()

    @pl.loop(0, _PER)
    def _(j):
        pltpu.async_copy(ones_v.at[0], hist_sh.at[idx_v.at[j]], sem_sc,
                         add=True)

    @pl.loop(0, _PER)
    def _(j):
        pltpu.make_async_copy(ones_v.at[0], hist_sh.at[idx_v.at[j]],
                              sem_sc).wait()

    plsc.subcore_barrier()

    sl = pl.ds(sid * _SLICE, _SLICE)

    @pl.when(cid == 0)
    def _():
        pltpu.sync_copy(hist_sh.at[sl], out0.at[sl])

    @pl.when(cid == 1)
    def _():
        pltpu.sync_copy(hist_sh.at[sl], out1.at[sl])


_ROWS = 1024


def _norm_block(x_ref, h0_ref, h1_ref, a_ref, o_ref, inv_ref):
    i = pl.program_id(0)

    @pl.when(i == 0)
    def _():
        deg = h0_ref[...] + h1_ref[...]
        a = a_ref[0]
        inv_ref[...] = jnp.exp(-a * jnp.log(deg + 1.0))

    off = pl.multiple_of(i * _ROWS, 128)
    inv = inv_ref[0:1, pl.ds(off, _ROWS)]
    n_col = jnp.reshape(inv, (_ROWS, 1))
    o_ref[...] = x_ref[...] * n_col


def _normalize(x, h0, h1, a_arr):
    return pl.pallas_call(
        _norm_block,
        grid=(pl.cdiv(_N_NODES, _ROWS),),
        in_specs=[
            pl.BlockSpec((_ROWS, _D), lambda i: (i, 0)),
            pl.BlockSpec((1, _PAD), lambda i: (0, 0)),
            pl.BlockSpec((1, _PAD), lambda i: (0, 0)),
            pl.BlockSpec(memory_space=pltpu.SMEM),
        ],
        out_specs=pl.BlockSpec((_ROWS, _D), lambda i: (i, 0)),
        out_shape=jax.ShapeDtypeStruct((_N_NODES, _D), jnp.float32),
        scratch_shapes=[pltpu.VMEM((1, _PAD), jnp.float32)],
    )(x, h0, h1, a_arr)


def kernel(x, edge_index, a):
    src = edge_index[0].reshape(_NTILES, _PER, _CHUNK)
    ones = jnp.ones((1, _CHUNK), jnp.float32)
    zeros = jnp.zeros((_SLICE,), jnp.float32)
    h0, h1 = _degree_hist(src, ones, zeros)
    a_arr = jnp.asarray(a, jnp.float32).reshape(1)
    return _normalize(x, h0.reshape(1, _PAD), h1.reshape(1, _PAD), a_arr)


# speedup vs baseline: 11.8958x; 1.5658x over previous
import functools

import jax
import jax.numpy as jnp
from jax import lax
from jax.experimental import pallas as pl
from jax.experimental.pallas import tpu as pltpu
from jax.experimental.pallas import tpu_sc as plsc

_N_NODES = 10000
_D = 128
_PAD = 10240
_CHUNK = 128
_NCHUNK = 2500
_NTILES = 32
_PER = 78
_SLICE = _PAD // 16

_mesh = plsc.VectorSubcoreMesh(core_axis_name="c", subcore_axis_name="s")


@functools.partial(
    pl.kernel,
    out_type=(
        jax.ShapeDtypeStruct((_PAD,), jnp.float32),
        jax.ShapeDtypeStruct((_PAD,), jnp.float32),
    ),
    mesh=_mesh,
    scratch_types=[
        pltpu.VMEM((_PER + 1, _CHUNK), jnp.int32),
        pltpu.VMEM((_CHUNK,), jnp.float32),
        pltpu.VMEM((_SLICE,), jnp.float32),
        pltpu.VMEM_SHARED((_PAD,), jnp.float32),
        pltpu.SemaphoreType.DMA,
        pltpu.SemaphoreType.DMA,
    ],
)
def _degree_hist(edges_hbm, out0, out1,
                 idx_v, ones_v, zer_v, hist_sh, sem_ld, sem_sc):
    cid = lax.axis_index("c")
    sid = lax.axis_index("s")
    w = cid * 16 + sid

    @pl.loop(0, _CHUNK // 16)
    def _(i):
        ones_v[pl.ds(i * 16, 16)] = jnp.full((16,), 1.0, jnp.float32)

    @pl.loop(0, _SLICE // 16)
    def _(i):
        zer_v[pl.ds(i * 16, 16)] = jnp.zeros((16,), jnp.float32)

    @pl.loop(0, _PER)
    def _(k):
        c = w + _NTILES * k
        pltpu.async_copy(edges_hbm.at[0, pl.ds(c * _CHUNK, _CHUNK)],
                         idx_v.at[k], sem_ld)

    @pl.when(w < _NCHUNK - _NTILES * _PER)
    def _():
        c = _NTILES * _PER + w
        pltpu.async_copy(edges_hbm.at[0, pl.ds(c * _CHUNK, _CHUNK)],
                         idx_v.at[_PER], sem_ld)

    pltpu.sync_copy(zer_v, hist_sh.at[pl.ds(sid * _SLICE, _SLICE)])
    plsc.subcore_barrier()

    @pl.loop(0, _PER)
    def _(k):
        c = w + _NTILES * k
        pltpu.make_async_copy(edges_hbm.at[0, pl.ds(c * _CHUNK, _CHUNK)],
                              idx_v.at[k], sem_ld).wait()

    @pl.when(w < _NCHUNK - _NTILES * _PER)
    def _():
        c = _NTILES * _PER + w
        pltpu.make_async_copy(edges_hbm.at[0, pl.ds(c * _CHUNK, _CHUNK)],
                              idx_v.at[_PER], sem_ld).wait()

    @pl.loop(0, _PER)
    def _(k):
        pltpu.async_copy(ones_v, hist_sh.at[idx_v.at[k]], sem_sc, add=True)

    @pl.when(w < _NCHUNK - _NTILES * _PER)
    def _():
        pltpu.async_copy(ones_v, hist_sh.at[idx_v.at[_PER]], sem_sc, add=True)

    @pl.loop(0, _PER)
    def _(k):
        pltpu.make_async_copy(ones_v, hist_sh.at[idx_v.at[k]], sem_sc).wait()

    @pl.when(w < _NCHUNK - _NTILES * _PER)
    def _():
        pltpu.make_async_copy(ones_v, hist_sh.at[idx_v.at[_PER]],
                              sem_sc).wait()

    plsc.subcore_barrier()

    sl = pl.ds(sid * _SLICE, _SLICE)

    @pl.when(cid == 0)
    def _():
        pltpu.sync_copy(hist_sh.at[sl], out0.at[sl])

    @pl.when(cid == 1)
    def _():
        pltpu.sync_copy(hist_sh.at[sl], out1.at[sl])


_ROWS = 1024


def _norm_block(x_ref, h0_ref, h1_ref, a_ref, o_ref, inv_ref):
    i = pl.program_id(0)

    @pl.when(i == 0)
    def _():
        deg = h0_ref[...] + h1_ref[...]
        a = a_ref[0]
        inv_ref[...] = jnp.exp(-a * jnp.log(deg + 1.0))

    off = pl.multiple_of(i * _ROWS, 128)
    inv = inv_ref[0:1, pl.ds(off, _ROWS)]
    n_col = jnp.reshape(inv, (_ROWS, 1))
    o_ref[...] = x_ref[...] * n_col


def _normalize(x, h0, h1, a_arr):
    return pl.pallas_call(
        _norm_block,
        grid=(pl.cdiv(_N_NODES, _ROWS),),
        in_specs=[
            pl.BlockSpec((_ROWS, _D), lambda i: (i, 0)),
            pl.BlockSpec((1, _PAD), lambda i: (0, 0)),
            pl.BlockSpec((1, _PAD), lambda i: (0, 0)),
            pl.BlockSpec(memory_space=pltpu.SMEM),
        ],
        out_specs=pl.BlockSpec((_ROWS, _D), lambda i: (i, 0)),
        out_shape=jax.ShapeDtypeStruct((_N_NODES, _D), jnp.float32),
        scratch_shapes=[pltpu.VMEM((1, _PAD), jnp.float32)],
    )(x, h0, h1, a_arr)


def kernel(x, edge_index, a):
    h0, h1 = _degree_hist(edge_index)
    a_arr = jnp.asarray(a, jnp.float32).reshape(1)
    return _normalize(x, h0.reshape(1, _PAD), h1.reshape(1, _PAD), a_arr)


# speedup vs baseline: 12.7798x; 1.0743x over previous
import functools

import jax
import jax.numpy as jnp
from jax import lax
from jax.experimental import pallas as pl
from jax.experimental.pallas import tpu as pltpu
from jax.experimental.pallas import tpu_sc as plsc

_N_NODES = 10000
_D = 128
_PAD = 10240
_CHUNK = 128
_NCHUNK = 2500
_NTILES = 32
_PER = 78
_SLICE = _PAD // 16

_mesh = plsc.VectorSubcoreMesh(core_axis_name="c", subcore_axis_name="s")


@functools.partial(
    pl.kernel,
    out_type=jax.ShapeDtypeStruct((2, _PAD), jnp.float32),
    mesh=_mesh,
    scratch_types=[
        pltpu.VMEM((_PER + 1, _CHUNK), jnp.int32),
        pltpu.VMEM((_CHUNK,), jnp.float32),
        pltpu.VMEM((_SLICE,), jnp.float32),
        pltpu.VMEM_SHARED((_PAD,), jnp.float32),
        pltpu.SemaphoreType.DMA,
        pltpu.SemaphoreType.DMA,
    ],
)
def _degree_hist(edges_hbm, out_hbm,
                 idx_v, ones_v, zer_v, hist_sh, sem_ld, sem_sc):
    cid = lax.axis_index("c")
    sid = lax.axis_index("s")
    w = cid * 16 + sid
    n_extra = _NCHUNK - _NTILES * _PER

    @pl.loop(0, _CHUNK // 16)
    def _(i):
        ones_v[pl.ds(i * 16, 16)] = jnp.full((16,), 1.0, jnp.float32)

    @pl.loop(0, _SLICE // 16)
    def _(i):
        zer_v[pl.ds(i * 16, 16)] = jnp.zeros((16,), jnp.float32)

    @pl.loop(0, _PER)
    def _(k):
        c = w + _NTILES * k
        pltpu.async_copy(edges_hbm.at[0, pl.ds(c * _CHUNK, _CHUNK)],
                         idx_v.at[k], sem_ld)

    @pl.when(w < n_extra)
    def _():
        c = _NTILES * _PER + w
        pltpu.async_copy(edges_hbm.at[0, pl.ds(c * _CHUNK, _CHUNK)],
                         idx_v.at[_PER], sem_ld)

    pltpu.sync_copy(zer_v, hist_sh.at[pl.ds(sid * _SLICE, _SLICE)])
    plsc.subcore_barrier()

    @pl.loop(0, _PER)
    def _(k):
        c = w + _NTILES * k
        pltpu.make_async_copy(edges_hbm.at[0, pl.ds(c * _CHUNK, _CHUNK)],
                              idx_v.at[k], sem_ld).wait()
        pltpu.async_copy(ones_v, hist_sh.at[idx_v.at[k]], sem_sc, add=True)

    @pl.when(w < n_extra)
    def _():
        c = _NTILES * _PER + w
        pltpu.make_async_copy(edges_hbm.at[0, pl.ds(c * _CHUNK, _CHUNK)],
                              idx_v.at[_PER], sem_ld).wait()
        pltpu.async_copy(ones_v, hist_sh.at[idx_v.at[_PER]], sem_sc, add=True)

    @pl.loop(0, _PER)
    def _(k):
        pltpu.make_async_copy(ones_v, hist_sh.at[idx_v.at[k]], sem_sc).wait()

    @pl.when(w < n_extra)
    def _():
        pltpu.make_async_copy(ones_v, hist_sh.at[idx_v.at[_PER]],
                              sem_sc).wait()

    plsc.subcore_barrier()

    sl = pl.ds(sid * _SLICE, _SLICE)
    pltpu.sync_copy(hist_sh.at[sl], out_hbm.at[cid, sl])


_ROWS = 2048


def _norm_block(x_ref, h_ref, a_ref, o_ref, inv_ref):
    i = pl.program_id(0)

    @pl.when(i == 0)
    def _():
        deg = h_ref[0:1, :] + h_ref[1:2, :]
        a = a_ref[0]
        inv_ref[...] = jnp.exp(-a * jnp.log(deg + 1.0))

    off = pl.multiple_of(i * _ROWS, 128)
    inv = inv_ref[0:1, pl.ds(off, _ROWS)]
    n_col = jnp.reshape(inv, (_ROWS, 1))
    o_ref[...] = x_ref[...] * n_col


def _normalize(x, hist, a_arr):
    return pl.pallas_call(
        _norm_block,
        grid=(pl.cdiv(_N_NODES, _ROWS),),
        in_specs=[
            pl.BlockSpec((_ROWS, _D), lambda i: (i, 0)),
            pl.BlockSpec((2, _PAD), lambda i: (0, 0)),
            pl.BlockSpec(memory_space=pltpu.SMEM),
        ],
        out_specs=pl.BlockSpec((_ROWS, _D), lambda i: (i, 0)),
        out_shape=jax.ShapeDtypeStruct((_N_NODES, _D), jnp.float32),
        scratch_shapes=[pltpu.VMEM((1, _PAD), jnp.float32)],
    )(x, hist, a_arr)


def kernel(x, edge_index, a):
    hist = _degree_hist(edge_index)
    a_arr = jnp.asarray(a, jnp.float32).reshape(1)
    return _normalize(x, hist, a_arr)
